# baseline (speedup 1.0000x reference)
import jax
import jax.numpy as jnp
from jax.experimental import pallas as pl
from jax.experimental.pallas import tpu as pltpu

N_KEEP = 1000
N_PAD = 1024
ROW_TILE = 8
COL_CHUNK = 128
N_RT = N_PAD // ROW_TILE
N_CC = N_PAD // COL_CHUNK


def _dense_kernel(do_mlp_ref,
                  boxes_r_ref, boxesT_ref, scores_r_ref, scoresT_ref,
                  classes_r_ref, classesT_ref,
                  W1_ref, b1_ref, W2_ref, b2_ref, W3_ref, b3_ref,
                  L1_ref, lb1_ref, L2_ref, lb2_ref,
                  out_ref, s_plane_ref):
    i = pl.program_id(0)

    boxes_r = boxes_r_ref[...]
    x1r = boxes_r[:, 0:1]
    y1r = boxes_r[:, 1:2]
    x2r = boxes_r[:, 2:3]
    y2r = boxes_r[:, 3:4]
    s_r = scores_r_ref[...]
    c_r = classes_r_ref[...]
    area_r = (x2r - x1r) * (y2r - y1r)

    row_ids = i * ROW_TILE + jax.lax.broadcasted_iota(
        jnp.int32, (ROW_TILE, 1), 0)

    W1 = W1_ref[...]
    b1 = b1_ref[...]
    W2 = W2_ref[...]
    b2 = b2_ref[...]
    W3 = W3_ref[...]
    b3 = b3_ref[...]

    s_plane_ref[...] = jnp.zeros((ROW_TILE, COL_CHUNK), jnp.float32)
    D_plane = jnp.zeros((ROW_TILE, COL_CHUNK), jnp.float32)

    for chunk in range(N_CC):
        c0 = chunk * COL_CHUNK
        x1c = boxesT_ref[0:1, c0:c0 + COL_CHUNK]
        y1c = boxesT_ref[1:2, c0:c0 + COL_CHUNK]
        x2c = boxesT_ref[2:3, c0:c0 + COL_CHUNK]
        y2c = boxesT_ref[3:4, c0:c0 + COL_CHUNK]
        s_c = scoresT_ref[0:1, c0:c0 + COL_CHUNK]
        area_c = (x2c - x1c) * (y2c - y1c)

        w = jnp.maximum(jnp.minimum(x2r, x2c) - jnp.maximum(x1r, x1c), 0.0)
        h = jnp.maximum(jnp.minimum(y2r, y2c) - jnp.maximum(y1r, y1c), 0.0)
        inter = w * h
        union = area_r + area_c - inter
        iou = inter / (union + 1e-06)

        col_ids = c0 + jax.lax.broadcasted_iota(
            jnp.int32, (1, COL_CHUNK), 1)
        diag = row_ids == col_ids
        iou = jnp.where(diag, 0.0, iou)

        D_plane = D_plane + iou

        @pl.when(do_mlp_ref[i, chunk] != 0)
        def _():
            c_c = classesT_ref[0:1, c0:c0 + COL_CHUNK]
            dx1 = jnp.abs(x1r - x1c)
            dy1 = jnp.abs(y1r - y1c)
            dx2 = jnp.abs(x2r - x2c)
            dy2 = jnp.abs(y2r - y2c)

            h2_acc = [None] * 16
            s_pre = None
            for k in range(32):
                rc = (b1[0, k] + W1[5, k] * s_r) + W1[6, k] * s_c
                h1k = jnp.maximum(
                    W1[0, k] * iou + W1[1, k] * dx1 + W1[2, k] * dy1
                    + W1[3, k] * dx2 + W1[4, k] * dy2 + rc, 0.0)
                for m in range(16):
                    t = W2[k, m] * h1k
                    h2_acc[m] = t if h2_acc[m] is None else h2_acc[m] + t
            for m in range(16):
                h2m = jnp.maximum(h2_acc[m] + b2[0, m], 0.0)
                t = W3[m, 0] * h2m
                s_pre = t if s_pre is None else s_pre + t
            s_ij = jax.nn.sigmoid(s_pre + b3[0, 0])

            mask = jnp.logical_and(c_r == c_c, s_c > s_r)
            contrib = jnp.where(mask, s_ij * iou, 0.0)
            s_plane_ref[...] += contrib

    D = jnp.sum(D_plane, axis=1, keepdims=True) * (1.0 / N_KEEP)
    S = jnp.sum(s_plane_ref[...], axis=1, keepdims=True)

    L1 = L1_ref[...]
    lb1 = lb1_ref[...]
    L2 = L2_ref[...]
    lb2 = lb2_ref[...]
    lam_cols = (x1r, y1r, x2r, y2r, s_r)
    lam_pre = None
    for t in range(16):
        a = lb1[0, t]
        for c in range(5):
            a = a + L1[c, t] * lam_cols[c]
        ht = jnp.maximum(a, 0.0)
        term = L2[t, 0] * ht
        lam_pre = term if lam_pre is None else lam_pre + term
    lam = jax.nn.sigmoid(lam_pre + lb2[0, 0])

    E = lam * S * D
    new_s = s_r * jnp.exp(-E)
    out_ref[...] = jnp.where(row_ids < N_KEEP, new_s, -1.0)


@jax.jit
def _dense_stage(boxes_k, scores_k, classes_k,
                 W1, b1, W2, b2, W3, b3, L1, lb1, L2, lb2):
    pad = N_PAD - N_KEEP
    boxes_p = jnp.pad(boxes_k, ((0, pad), (0, 0)))
    scores_p = jnp.pad(scores_k, (0, pad), constant_values=-1.0)
    classes_p = jnp.pad(classes_k, (0, pad), constant_values=127)

    cls_first = classes_p[::ROW_TILE]
    cls_last = classes_p[ROW_TILE - 1::ROW_TILE]
    ws = jnp.searchsorted(classes_p, cls_first, side="left")
    we = jnp.searchsorted(classes_p, cls_last, side="right")
    c_lo = jnp.arange(N_CC) * COL_CHUNK
    c_hi = c_lo + COL_CHUNK
    do_mlp = jnp.logical_and(c_hi[None, :] > ws[:, None],
                             c_lo[None, :] < we[:, None]).astype(jnp.int32)

    boxesT = boxes_p.T
    scores_r = scores_p[:, None]
    scoresT = scores_p[None, :]
    classes_r = classes_p[:, None]
    classesT = classes_p[None, :]

    grid = (N_RT,)
    row_spec2 = lambda w: pl.BlockSpec((ROW_TILE, w), lambda i: (i, 0))
    full = lambda a, b: pl.BlockSpec((a, b), lambda i: (0, 0))
    smem = pl.BlockSpec(memory_space=pltpu.SMEM)

    out = pl.pallas_call(
        _dense_kernel,
        grid=grid,
        in_specs=[
            smem,
            row_spec2(4),
            full(4, N_PAD),
            row_spec2(1),
            full(1, N_PAD),
            row_spec2(1),
            full(1, N_PAD),
            full(7, 32), full(1, 32),
            full(32, 16), full(1, 16),
            full(16, 1), full(1, 1),
            full(5, 16), full(1, 16),
            full(16, 1), full(1, 1),
        ],
        out_specs=pl.BlockSpec((ROW_TILE, 1), lambda i: (i, 0)),
        out_shape=jax.ShapeDtypeStruct((N_PAD, 1), jnp.float32),
        scratch_shapes=[pltpu.VMEM((ROW_TILE, COL_CHUNK), jnp.float32)],
    )(do_mlp, boxes_p, boxesT, scores_r, scoresT, classes_r, classesT,
      W1, b1[None, :], W2, b2[None, :], W3, b3[None, :],
      L1, lb1[None, :], L2, lb2[None, :])
    return out[:N_KEEP, 0]


def kernel(boxes, scores, classes, W1, b1, W2, b2, W3, b3, L1, lb1, L2, lb2):
    scores_k, idx = jax.lax.top_k(scores, N_KEEP)
    boxes_k = boxes[idx]
    classes_k = classes[idx]

    perm = jnp.argsort(classes_k, stable=True)
    boxes_s = boxes_k[perm]
    scores_s = scores_k[perm]
    classes_s = classes_k[perm]

    new_scores = _dense_stage(boxes_s, scores_s, classes_s,
                              W1, b1, W2, b2, W3, b3, L1, lb1, L2, lb2)
    _, idx2 = jax.lax.top_k(new_scores, 50)
    return (boxes_s[idx2], new_scores[idx2], classes_s[idx2])


# speedup vs baseline: 4.7416x; 4.7416x over previous
import jax
import jax.numpy as jnp
from jax.experimental import pallas as pl
from jax.experimental.pallas import tpu as pltpu

N_KEEP = 1000
N_PAD = 1024
ROW_TILE = 16
COL_CHUNK = 128
N_RT = N_PAD // ROW_TILE
N_CC = N_PAD // COL_CHUNK

OFF_W1 = 0
OFF_B1 = 224
OFF_W2 = 256
OFF_B2 = 768
OFF_W3 = 784
OFF_B3 = 800
OFF_L1 = 801
OFF_LB1 = 881
OFF_L2 = 897
OFF_LB2 = 913
N_TBL = 914


def _dense_kernel(do_mlp_ref,
                  boxes_r_ref, boxesT_ref, scores_r_ref, scoresT_ref,
                  classes_r_ref, classesT_ref, tbl_ref,
                  out_ref, s_plane_ref):
    i = pl.program_id(0)

    ones = jnp.ones((ROW_TILE, COL_CHUNK), jnp.float32)
    x1r = boxes_r_ref[:, 0:1] * ones
    y1r = boxes_r_ref[:, 1:2] * ones
    x2r = boxes_r_ref[:, 2:3] * ones
    y2r = boxes_r_ref[:, 3:4] * ones
    s_r = scores_r_ref[...] * ones
    c_r = classes_r_ref[...] * jnp.ones((ROW_TILE, COL_CHUNK), jnp.int32)
    area_r = (x2r - x1r) * (y2r - y1r)

    row_ids = i * ROW_TILE + jax.lax.broadcasted_iota(
        jnp.int32, (ROW_TILE, COL_CHUNK), 0)

    def wt(idx):
        return tbl_ref[idx]

    s_plane_ref[...] = jnp.zeros((ROW_TILE, COL_CHUNK), jnp.float32)
    D_plane = jnp.zeros((ROW_TILE, COL_CHUNK), jnp.float32)

    for chunk in range(N_CC):
        c0 = chunk * COL_CHUNK
        x1c = boxesT_ref[0:1, c0:c0 + COL_CHUNK] * ones
        y1c = boxesT_ref[1:2, c0:c0 + COL_CHUNK] * ones
        x2c = boxesT_ref[2:3, c0:c0 + COL_CHUNK] * ones
        y2c = boxesT_ref[3:4, c0:c0 + COL_CHUNK] * ones
        s_c = scoresT_ref[0:1, c0:c0 + COL_CHUNK] * ones
        area_c = (x2c - x1c) * (y2c - y1c)

        w = jnp.maximum(jnp.minimum(x2r, x2c) - jnp.maximum(x1r, x1c), 0.0)
        h = jnp.maximum(jnp.minimum(y2r, y2c) - jnp.maximum(y1r, y1c), 0.0)
        inter = w * h
        union = area_r + area_c - inter
        iou = inter / (union + 1e-06)

        col_ids = c0 + jax.lax.broadcasted_iota(
            jnp.int32, (ROW_TILE, COL_CHUNK), 1)
        iou = jnp.where(row_ids == col_ids, 0.0, iou)

        D_plane = D_plane + iou

        @pl.when(do_mlp_ref[i, chunk] != 0)
        def _():
            c_c = classesT_ref[0:1, c0:c0 + COL_CHUNK] * jnp.ones(
                (ROW_TILE, COL_CHUNK), jnp.int32)
            dx1 = jnp.abs(x1r - x1c)
            dy1 = jnp.abs(y1r - y1c)
            dx2 = jnp.abs(x2r - x2c)
            dy2 = jnp.abs(y2r - y2c)

            h2 = [wt(OFF_B2 + m) for m in range(16)]
            for k in range(32):
                t = wt(OFF_B1 + k)
                t = t + wt(OFF_W1 + 0 * 32 + k) * iou
                t = t + wt(OFF_W1 + 1 * 32 + k) * dx1
                t = t + wt(OFF_W1 + 2 * 32 + k) * dy1
                t = t + wt(OFF_W1 + 3 * 32 + k) * dx2
                t = t + wt(OFF_W1 + 4 * 32 + k) * dy2
                t = t + wt(OFF_W1 + 5 * 32 + k) * s_r
                t = t + wt(OFF_W1 + 6 * 32 + k) * s_c
                h1k = jnp.maximum(t, 0.0)
                for m in range(16):
                    h2[m] = h2[m] + wt(OFF_W2 + k * 16 + m) * h1k
            s_pre = wt(OFF_B3)
            for m in range(16):
                s_pre = s_pre + wt(OFF_W3 + m) * jnp.maximum(h2[m], 0.0)
            s_ij = jax.nn.sigmoid(s_pre)

            mask = jnp.logical_and(c_r == c_c, s_c > s_r)
            contrib = jnp.where(mask, s_ij * iou, 0.0)
            s_plane_ref[...] += contrib

    D = jnp.sum(D_plane, axis=1, keepdims=True) * (1.0 / N_KEEP)
    S = jnp.sum(s_plane_ref[...], axis=1, keepdims=True)

    lam_cols = (x1r, y1r, x2r, y2r, s_r)
    lam_pre = wt(OFF_LB2)
    for t in range(16):
        a = wt(OFF_LB1 + t)
        for c in range(5):
            a = a + wt(OFF_L1 + c * 16 + t) * lam_cols[c]
        lam_pre = lam_pre + wt(OFF_L2 + t) * jnp.maximum(a, 0.0)
    lam = jax.nn.sigmoid(lam_pre[:, 0:1])

    E = lam * S * D
    new_s = scores_r_ref[...] * jnp.exp(-E)
    out_ref[...] = jnp.where(row_ids[:, 0:1] < N_KEEP, new_s, -1.0)


@jax.jit
def _dense_stage(boxes_k, scores_k, classes_k,
                 W1, b1, W2, b2, W3, b3, L1, lb1, L2, lb2):
    pad = N_PAD - N_KEEP
    boxes_p = jnp.pad(boxes_k, ((0, pad), (0, 0)))
    scores_p = jnp.pad(scores_k, (0, pad), constant_values=-1.0)
    classes_p = jnp.pad(classes_k, (0, pad), constant_values=127)

    cls_first = classes_p[::ROW_TILE]
    cls_last = classes_p[ROW_TILE - 1::ROW_TILE]
    ws = jnp.searchsorted(classes_p, cls_first, side="left")
    we = jnp.searchsorted(classes_p, cls_last, side="right")
    c_lo = jnp.arange(N_CC) * COL_CHUNK
    c_hi = c_lo + COL_CHUNK
    do_mlp = jnp.logical_and(c_hi[None, :] > ws[:, None],
                             c_lo[None, :] < we[:, None]).astype(jnp.int32)

    vals = jnp.concatenate([
        W1.reshape(-1), b1, W2.reshape(-1), b2, W3.reshape(-1), b3,
        L1.reshape(-1), lb1, L2.reshape(-1), lb2])
    tbl = jnp.broadcast_to(vals[:, None, None],
                           (N_TBL, ROW_TILE, COL_CHUNK))

    boxesT = boxes_p.T
    scores_r = scores_p[:, None]
    scoresT = scores_p[None, :]
    classes_r = classes_p[:, None]
    classesT = classes_p[None, :]

    grid = (N_RT,)
    row_spec2 = lambda w: pl.BlockSpec((ROW_TILE, w), lambda i: (i, 0))
    full = lambda a, b: pl.BlockSpec((a, b), lambda i: (0, 0))
    smem = pl.BlockSpec(memory_space=pltpu.SMEM)

    out = pl.pallas_call(
        _dense_kernel,
        grid=grid,
        in_specs=[
            smem,
            row_spec2(4),
            full(4, N_PAD),
            row_spec2(1),
            full(1, N_PAD),
            row_spec2(1),
            full(1, N_PAD),
            pl.BlockSpec((N_TBL, ROW_TILE, COL_CHUNK),
                         lambda i: (0, 0, 0)),
        ],
        out_specs=pl.BlockSpec((ROW_TILE, 1), lambda i: (i, 0)),
        out_shape=jax.ShapeDtypeStruct((N_PAD, 1), jnp.float32),
        scratch_shapes=[pltpu.VMEM((ROW_TILE, COL_CHUNK), jnp.float32)],
    )(do_mlp, boxes_p, boxesT, scores_r, scoresT, classes_r, classesT, tbl)
    return out[:N_KEEP, 0]


def kernel(boxes, scores, classes, W1, b1, W2, b2, W3, b3, L1, lb1, L2, lb2):
    scores_k, idx = jax.lax.top_k(scores, N_KEEP)
    boxes_k = boxes[idx]
    classes_k = classes[idx]

    perm = jnp.argsort(classes_k, stable=True)
    boxes_s = boxes_k[perm]
    scores_s = scores_k[perm]
    classes_s = classes_k[perm]

    new_scores = _dense_stage(boxes_s, scores_s, classes_s,
                              W1, b1, W2, b2, W3, b3, L1, lb1, L2, lb2)
    _, idx2 = jax.lax.top_k(new_scores, 50)
    return (boxes_s[idx2], new_scores[idx2], classes_s[idx2])
